# Initial kernel scaffold; baseline (speedup 1.0000x reference)
import jax
import jax.numpy as jnp
from jax.experimental import pallas as pl


def kernel(idx, weight):
    raise NotImplementedError("write your pallas kernel here")


# speedup vs baseline: 1.5685x; 1.5685x over previous
import functools

import jax
import jax.numpy as jnp
from jax import lax
from jax.experimental import pallas as pl
from jax.experimental.pallas import tpu as pltpu
from jax.experimental.pallas import tpu_sc as plsc

NUM_ROWS = 16384 * 26
DIM = 32
NUM_WORKERS = 32
ROWS_PER_WORKER = NUM_ROWS // NUM_WORKERS
CHUNK = 1664
NUM_CHUNKS = ROWS_PER_WORKER // CHUNK

_mesh = plsc.VectorSubcoreMesh(core_axis_name="c", subcore_axis_name="s")


@functools.partial(
    pl.kernel,
    out_type=jax.ShapeDtypeStruct((NUM_ROWS, DIM), jnp.float32),
    mesh=_mesh,
    scratch_types=[
        pltpu.VMEM((ROWS_PER_WORKER,), jnp.int32),
        pltpu.VMEM((2, CHUNK, DIM), jnp.float32),
        pltpu.SemaphoreType.DMA,
    ],
    compiler_params=pltpu.CompilerParams(use_tc_tiling_on_sc=False),
)
def _embed_sc(idx_hbm, table_hbm, out_hbm, idx_v, rows_v, gsem):
    wid = lax.axis_index("s") * 2 + lax.axis_index("c")
    base = wid * ROWS_PER_WORKER
    pltpu.sync_copy(idx_hbm.at[pl.ds(base, ROWS_PER_WORKER)], idx_v)

    copies = [None, None]
    copies[0] = pltpu.async_copy(
        table_hbm.at[idx_v.at[pl.ds(0, CHUNK)]], rows_v.at[0], gsem
    )
    for c in range(NUM_CHUNKS):
        copies[c % 2].wait()
        if c + 1 < NUM_CHUNKS:
            copies[(c + 1) % 2] = pltpu.async_copy(
                table_hbm.at[idx_v.at[pl.ds((c + 1) * CHUNK, CHUNK)]],
                rows_v.at[(c + 1) % 2],
                gsem,
            )
        pltpu.sync_copy(rows_v.at[c % 2], out_hbm.at[pl.ds(base + c * CHUNK, CHUNK)])


def kernel(idx, weight):
    idx_flat = idx.reshape(-1).astype(jnp.int32)
    out = _embed_sc(idx_flat, weight)
    return out.reshape(idx.shape + (DIM,))


# speedup vs baseline: 1.5765x; 1.0051x over previous
import functools

import jax
import jax.numpy as jnp
from jax import lax
from jax.experimental import pallas as pl
from jax.experimental.pallas import tpu as pltpu
from jax.experimental.pallas import tpu_sc as plsc

NUM_ROWS = 16384 * 26
DIM = 32
NUM_WORKERS = 32
ROWS_PER_WORKER = NUM_ROWS // NUM_WORKERS
CHUNK = 832
NUM_CHUNKS = ROWS_PER_WORKER // CHUNK
NBUF = 4
INFLIGHT = 3

_mesh = plsc.VectorSubcoreMesh(core_axis_name="c", subcore_axis_name="s")


@functools.partial(
    pl.kernel,
    out_type=jax.ShapeDtypeStruct((NUM_ROWS, DIM), jnp.float32),
    mesh=_mesh,
    scratch_types=[
        pltpu.VMEM((ROWS_PER_WORKER,), jnp.int32),
        pltpu.VMEM((NBUF, CHUNK, DIM), jnp.float32),
        pltpu.SemaphoreType.DMA,
        pltpu.SemaphoreType.DMA,
    ],
    compiler_params=pltpu.CompilerParams(use_tc_tiling_on_sc=False),
)
def _embed_sc(idx_hbm, table_hbm, out_hbm, idx_v, rows_v, gsem, ssem):
    wid = lax.axis_index("s") * 2 + lax.axis_index("c")
    base = wid * ROWS_PER_WORKER
    pltpu.sync_copy(idx_hbm.at[pl.ds(base, ROWS_PER_WORKER)], idx_v)

    def gather(c):
        return pltpu.async_copy(
            table_hbm.at[idx_v.at[pl.ds(c * CHUNK, CHUNK)]],
            rows_v.at[c % NBUF],
            gsem,
        )

    gath = [None] * NUM_CHUNKS
    stor = [None] * NUM_CHUNKS
    for c in range(INFLIGHT):
        gath[c] = gather(c)
    for c in range(NUM_CHUNKS):
        if c + INFLIGHT < NUM_CHUNKS:
            if c >= NBUF - INFLIGHT:
                stor[c + INFLIGHT - NBUF].wait()
            gath[c + INFLIGHT] = gather(c + INFLIGHT)
        gath[c].wait()
        stor[c] = pltpu.async_copy(
            rows_v.at[c % NBUF], out_hbm.at[pl.ds(base + c * CHUNK, CHUNK)], ssem
        )
    for c in range(NUM_CHUNKS - NBUF, NUM_CHUNKS):
        stor[c].wait()


def kernel(idx, weight):
    idx_flat = idx.reshape(-1).astype(jnp.int32)
    out = _embed_sc(idx_flat, weight)
    return out.reshape(idx.shape + (DIM,))
